# Initial kernel scaffold; baseline (speedup 1.0000x reference)
import jax
import jax.numpy as jnp
from jax.experimental import pallas as pl


def kernel(f_token_ids, crit_emb):
    raise NotImplementedError("write your pallas kernel here")


# speedup vs baseline: 73.8266x; 73.8266x over previous
import functools

import jax
import jax.numpy as jnp
from jax import lax
from jax.experimental import pallas as pl
from jax.experimental.pallas import tpu as pltpu
from jax.experimental.pallas import tpu_sc as plsc


def _sc_histogram(ids_flat, B, L, V):
    info = plsc.get_sparse_core_info()
    NC, NS, LANES = info.num_cores, info.num_subcores, info.num_lanes
    NW = NC * NS

    rows_per_worker = B // NW
    rows_per_stage = 128
    num_stages = rows_per_worker // rows_per_stage
    subs = rows_per_stage // LANES

    mesh = plsc.VectorSubcoreMesh(core_axis_name="c", subcore_axis_name="s")

    @functools.partial(
        pl.kernel,
        out_type=jax.ShapeDtypeStruct((B * V,), jnp.float32),
        mesh=mesh,
        scratch_types=[
            pltpu.VMEM((rows_per_stage * L,), jnp.int32),
            pltpu.VMEM((rows_per_stage * V,), jnp.float32),
        ],
        compiler_params=pltpu.CompilerParams(needs_layout_passes=False),
    )
    def hist(ids_hbm, counts_hbm, ids_v, counts_v):
        wid = lax.axis_index("s") * NC + lax.axis_index("c")
        iota = lax.iota(jnp.int32, LANES)
        ones = jnp.ones((LANES,), jnp.float32)
        zeros = jnp.zeros((LANES,), jnp.float32)

        for stage in range(num_stages):
            r0 = wid * rows_per_worker + stage * rows_per_stage
            pltpu.sync_copy(ids_hbm.at[pl.ds(r0 * L, rows_per_stage * L)], ids_v)

            def zbody(i, _):
                counts_v[pl.ds(i * LANES, LANES)] = zeros
                return 0

            lax.fori_loop(0, rows_per_stage * V // LANES, zbody, 0)

            for sub in range(subs):
                lane_rows = sub * LANES + iota
                idx0 = lane_rows * L
                rowoff = lane_rows * V

                def lbody(l, idx):
                    ids16 = plsc.load_gather(ids_v, [idx])
                    plsc.addupdate_scatter(counts_v, [rowoff + ids16], ones)
                    return idx + 1

                lax.fori_loop(0, L, lbody, idx0)

            pltpu.sync_copy(counts_v, counts_hbm.at[pl.ds(r0 * V, rows_per_stage * V)])

    return hist(ids_flat)


def _tc_matmul(counts, emb, L):
    B, V = counts.shape
    _, D = emb.shape
    BM = 2048
    inv_l = 1.0 / float(L)

    def body(c_ref, e_ref, o_ref):
        o_ref[...] = (
            jnp.dot(c_ref[...], e_ref[...], preferred_element_type=jnp.float32)
            * inv_l
        )

    return pl.pallas_call(
        body,
        grid=(B // BM,),
        in_specs=[
            pl.BlockSpec((BM, V), lambda i: (i, 0)),
            pl.BlockSpec((V, D), lambda i: (0, 0)),
        ],
        out_specs=pl.BlockSpec((BM, D), lambda i: (i, 0)),
        out_shape=jax.ShapeDtypeStruct((B, D), jnp.float32),
    )(counts, emb)


def kernel(f_token_ids, crit_emb):
    B, L = f_token_ids.shape
    V, D = crit_emb.shape
    counts_flat = _sc_histogram(f_token_ids.reshape(-1), B, L, V)
    counts = counts_flat.reshape(B, V)
    return _tc_matmul(counts, crit_emb, L)


# speedup vs baseline: 80.9992x; 1.0972x over previous
import functools

import jax
import jax.numpy as jnp
from jax import lax
from jax.experimental import pallas as pl
from jax.experimental.pallas import tpu as pltpu
from jax.experimental.pallas import tpu_sc as plsc

_UNROLL = 8
_ZUNROLL = 16


def _sc_histogram(ids, B, L, V):
    info = plsc.get_sparse_core_info()
    NC, NS, LANES = info.num_cores, info.num_subcores, info.num_lanes
    NW = NC * NS

    rows_per_worker = B // NW
    rows_per_stage = 128
    num_stages = rows_per_worker // rows_per_stage
    subs = rows_per_stage // LANES

    mesh = plsc.VectorSubcoreMesh(core_axis_name="c", subcore_axis_name="s")

    @functools.partial(
        pl.kernel,
        out_type=jax.ShapeDtypeStruct((B, V), jnp.float32),
        mesh=mesh,
        scratch_types=[
            pltpu.VMEM((rows_per_stage, L), jnp.int32),
            pltpu.VMEM((rows_per_stage, V), jnp.float32),
        ],
        compiler_params=pltpu.CompilerParams(needs_layout_passes=False),
    )
    def hist(ids_hbm, counts_hbm, ids_v, counts_v):
        wid = lax.axis_index("s") * NC + lax.axis_index("c")
        iota = lax.iota(jnp.int32, LANES)
        ones = jnp.ones((LANES,), jnp.float32)
        zeros = jnp.zeros((LANES,), jnp.float32)

        for stage in range(num_stages):
            r0 = wid * rows_per_worker + stage * rows_per_stage
            pltpu.sync_copy(ids_hbm.at[pl.ds(r0, rows_per_stage)], ids_v)

            def zbody(i, _):
                for u in range(_ZUNROLL):
                    j = i * (_ZUNROLL * LANES) + u * LANES
                    r = j // V
                    c = j % V
                    counts_v[r, pl.ds(c, LANES)] = zeros
                return 0

            lax.fori_loop(0, rows_per_stage * V // (LANES * _ZUNROLL), zbody, 0)

            for sub in range(subs):
                rows16 = sub * LANES + iota

                def lbody(i, lsplat):
                    for u in range(_UNROLL):
                        ids16 = plsc.load_gather(ids_v, [rows16, lsplat + u])
                        plsc.addupdate_scatter(counts_v, [rows16, ids16], ones)
                    return lsplat + _UNROLL

                lax.fori_loop(0, L // _UNROLL, lbody, jnp.zeros((LANES,), jnp.int32))

            pltpu.sync_copy(counts_v, counts_hbm.at[pl.ds(r0, rows_per_stage)])

    return hist(ids)


def _tc_matmul(counts, emb, L):
    B, V = counts.shape
    _, D = emb.shape
    BM = 2048
    inv_l = 1.0 / float(L)

    def body(c_ref, e_ref, o_ref):
        o_ref[...] = (
            jnp.dot(c_ref[...], e_ref[...], preferred_element_type=jnp.float32)
            * inv_l
        )

    return pl.pallas_call(
        body,
        grid=(B // BM,),
        in_specs=[
            pl.BlockSpec((BM, V), lambda i: (i, 0)),
            pl.BlockSpec((V, D), lambda i: (0, 0)),
        ],
        out_specs=pl.BlockSpec((BM, D), lambda i: (i, 0)),
        out_shape=jax.ShapeDtypeStruct((B, D), jnp.float32),
    )(counts, emb)


def kernel(f_token_ids, crit_emb):
    B, L = f_token_ids.shape
    V, D = crit_emb.shape
    counts = _sc_histogram(f_token_ids, B, L, V)
    return _tc_matmul(counts, crit_emb, L)


# speedup vs baseline: 109.2213x; 1.3484x over previous
import functools

import jax
import jax.numpy as jnp
from jax import lax
from jax.experimental import pallas as pl
from jax.experimental.pallas import tpu as pltpu
from jax.experimental.pallas import tpu_sc as plsc

_UNROLL = 8
_ZUNROLL = 16


def _sc_histogram(ids, B, L, V):
    info = plsc.get_sparse_core_info()
    NC, NS, LANES = info.num_cores, info.num_subcores, info.num_lanes
    NW = NC * NS

    HALF = V // 2
    rows_per_worker = B // NW
    rows_per_stage = 128
    num_stages = rows_per_worker // rows_per_stage
    subs = rows_per_stage // LANES
    slab = rows_per_stage * HALF

    mesh = plsc.VectorSubcoreMesh(core_axis_name="c", subcore_axis_name="s")

    @functools.partial(
        pl.kernel,
        out_type=jax.ShapeDtypeStruct((2 * B * HALF,), jnp.float32),
        mesh=mesh,
        scratch_types=[
            pltpu.VMEM((rows_per_stage, L), jnp.int32),
            pltpu.VMEM((2 * slab,), jnp.float32),
        ],
        compiler_params=pltpu.CompilerParams(needs_layout_passes=False),
    )
    def hist(ids_hbm, counts_hbm, ids_v, counts_v):
        wid = lax.axis_index("s") * NC + lax.axis_index("c")
        iota = lax.iota(jnp.int32, LANES)
        ones = jnp.ones((LANES,), jnp.float32)
        zeros = jnp.zeros((LANES,), jnp.float32)

        for stage in range(num_stages):
            r0 = wid * rows_per_worker + stage * rows_per_stage
            pltpu.sync_copy(ids_hbm.at[pl.ds(r0, rows_per_stage)], ids_v)

            def zbody(i, _):
                for u in range(_ZUNROLL):
                    counts_v[pl.ds(i * (_ZUNROLL * LANES) + u * LANES, LANES)] = zeros
                return 0

            lax.fori_loop(0, 2 * slab // (LANES * _ZUNROLL), zbody, 0)

            for sub in range(subs):
                rows16 = sub * LANES + iota
                rowbase = rows16 * HALF

                def lbody(i, lsplat):
                    toks = [
                        plsc.load_gather(ids_v, [rows16, lsplat + u])
                        for u in range(_UNROLL)
                    ]
                    for t in toks:
                        addr = rowbase + (t & 0x7F) + ((t & 0x80) << 7)
                        plsc.addupdate_scatter(counts_v, [addr], ones)
                    return lsplat + _UNROLL

                lax.fori_loop(0, L // _UNROLL, lbody, jnp.zeros((LANES,), jnp.int32))

            pltpu.sync_copy(
                counts_v.at[pl.ds(0, slab)],
                counts_hbm.at[pl.ds(r0 * HALF, slab)],
            )
            pltpu.sync_copy(
                counts_v.at[pl.ds(slab, slab)],
                counts_hbm.at[pl.ds(B * HALF + r0 * HALF, slab)],
            )

    return hist(ids)


def _tc_matmul(c_lo, c_hi, e_lo, e_hi, L):
    B, HALF = c_lo.shape
    _, D = e_lo.shape
    BM = 2048
    inv_l = 1.0 / float(L)

    def body(cl_ref, ch_ref, el_ref, eh_ref, o_ref):
        acc = jnp.dot(cl_ref[...], el_ref[...], preferred_element_type=jnp.float32)
        acc += jnp.dot(ch_ref[...], eh_ref[...], preferred_element_type=jnp.float32)
        o_ref[...] = acc * inv_l

    return pl.pallas_call(
        body,
        grid=(B // BM,),
        in_specs=[
            pl.BlockSpec((BM, HALF), lambda i: (i, 0)),
            pl.BlockSpec((BM, HALF), lambda i: (i, 0)),
            pl.BlockSpec((HALF, D), lambda i: (0, 0)),
            pl.BlockSpec((HALF, D), lambda i: (0, 0)),
        ],
        out_specs=pl.BlockSpec((BM, D), lambda i: (i, 0)),
        out_shape=jax.ShapeDtypeStruct((B, D), jnp.float32),
    )(c_lo, c_hi, e_lo, e_hi)


def kernel(f_token_ids, crit_emb):
    B, L = f_token_ids.shape
    V, D = crit_emb.shape
    HALF = V // 2
    counts_flat = _sc_histogram(f_token_ids, B, L, V)
    c_lo = counts_flat[: B * HALF].reshape(B, HALF)
    c_hi = counts_flat[B * HALF :].reshape(B, HALF)
    return _tc_matmul(c_lo, c_hi, crit_emb[:HALF], crit_emb[HALF:], L)


# speedup vs baseline: 122.3854x; 1.1205x over previous
import functools

import jax
import jax.numpy as jnp
from jax import lax
from jax.experimental import pallas as pl
from jax.experimental.pallas import tpu as pltpu
from jax.experimental.pallas import tpu_sc as plsc

_UNROLL = 8
_ZUNROLL = 16


def _sc_histogram(ids, B, L, V):
    info = plsc.get_sparse_core_info()
    NC, NS, LANES = info.num_cores, info.num_subcores, info.num_lanes
    NW = NC * NS

    HALF = V // 2
    rows_per_worker = B // NW
    rows_per_stage = 128
    num_stages = rows_per_worker // rows_per_stage
    subs = rows_per_stage // LANES
    slab = rows_per_stage * HALF

    mesh = plsc.VectorSubcoreMesh(core_axis_name="c", subcore_axis_name="s")

    @functools.partial(
        pl.kernel,
        out_type=jax.ShapeDtypeStruct((2 * B * HALF,), jnp.float32),
        mesh=mesh,
        scratch_types=[
            pltpu.VMEM((rows_per_stage, L), jnp.int32),
            pltpu.VMEM((2 * slab,), jnp.float32),
        ],
        compiler_params=pltpu.CompilerParams(needs_layout_passes=False),
    )
    def hist(ids_hbm, counts_hbm, ids_v, counts_v):
        wid = lax.axis_index("s") * NC + lax.axis_index("c")
        iota = lax.iota(jnp.int32, LANES)
        ones = jnp.ones((LANES,), jnp.float32)
        zeros = jnp.zeros((LANES,), jnp.float32)

        for stage in range(num_stages):
            r0 = wid * rows_per_worker + stage * rows_per_stage
            pltpu.sync_copy(ids_hbm.at[pl.ds(r0, rows_per_stage)], ids_v)

            def zbody(i, _):
                for u in range(_ZUNROLL):
                    counts_v[pl.ds(i * (_ZUNROLL * LANES) + u * LANES, LANES)] = zeros
                return 0

            lax.fori_loop(0, 2 * slab // (LANES * _ZUNROLL), zbody, 0)

            for sub in range(subs):
                rows16 = sub * LANES + iota
                rowbase = rows16 * HALF

                def lbody(i, lsplat):
                    toks = [
                        plsc.load_gather(ids_v, [rows16, lsplat + u])
                        for u in range(_UNROLL)
                    ]
                    for t in toks:
                        addr = rowbase + (t & 0x7F) + ((t & 0x80) << 7)
                        plsc.addupdate_scatter(counts_v, [addr], ones)
                    return lsplat + _UNROLL

                lax.fori_loop(0, L // _UNROLL, lbody, jnp.zeros((LANES,), jnp.int32))

            pltpu.sync_copy(
                counts_v.at[pl.ds(0, slab)],
                counts_hbm.at[pl.ds(r0 * HALF, slab)],
            )
            pltpu.sync_copy(
                counts_v.at[pl.ds(slab, slab)],
                counts_hbm.at[pl.ds(B * HALF + r0 * HALF, slab)],
            )

    return hist(ids)


def _tc_matmul(counts3, emb, L):
    _, B, HALF = counts3.shape
    V, D = emb.shape
    BM = 2048
    inv_l = 1.0 / float(L)

    def body(c_ref, e_ref, o_ref):
        acc = jnp.dot(c_ref[0], e_ref[:HALF], preferred_element_type=jnp.float32)
        acc += jnp.dot(c_ref[1], e_ref[HALF:], preferred_element_type=jnp.float32)
        o_ref[...] = acc * inv_l

    return pl.pallas_call(
        body,
        grid=(B // BM,),
        in_specs=[
            pl.BlockSpec((2, BM, HALF), lambda i: (0, i, 0)),
            pl.BlockSpec((V, D), lambda i: (0, 0)),
        ],
        out_specs=pl.BlockSpec((BM, D), lambda i: (i, 0)),
        out_shape=jax.ShapeDtypeStruct((B, D), jnp.float32),
    )(counts3, emb)


def kernel(f_token_ids, crit_emb):
    B, L = f_token_ids.shape
    V, D = crit_emb.shape
    HALF = V // 2
    counts_flat = _sc_histogram(f_token_ids, B, L, V)
    counts3 = counts_flat.reshape(2, B, HALF)
    return _tc_matmul(counts3, crit_emb, L)
